# baseline (speedup 1.0000x reference)
import functools

import jax
import jax.numpy as jnp
from jax import lax
from jax.experimental import pallas as pl
from jax.experimental.pallas import tpu as pltpu


def _fused_kernel(
    af_ref,
    z1_ref,
    wm_ref,
    bm_ref,
    glow_ref,
    w1_ref,
    w2_ref,
    b12_ref,
    w3_ref,
    b3_ref,
    o_ref,
    *,
    t_len,
    n_chunks,
):
    R = z1_ref.shape[0]
    cin = z1_ref.shape[1]
    hid = w2_ref.shape[0]
    cout = o_ref.shape[1]
    rc = R // n_chunks
    nbc = rc // t_len

    t_idx = lax.broadcasted_iota(jnp.int32, (rc, 1), 0) % t_len
    is_first = t_idx == 0
    is_last = t_idx == (t_len - 1)

    w1z = jnp.concatenate(
        [w1_ref[0:cin, :], w1_ref[2 * cin:3 * cin, :], w1_ref[4 * cin:5 * cin, :]],
        axis=0).astype(jnp.bfloat16)
    w2 = w2_ref[...].astype(jnp.bfloat16)
    w3 = jnp.concatenate(
        [w3_ref[0:hid, :], w3_ref[hid:2 * hid, :], w3_ref[2 * hid:, :]],
        axis=-1).astype(jnp.bfloat16)
    b12 = b12_ref[...].astype(jnp.bfloat16)

    w1af = jnp.concatenate(
        [w1_ref[cin:2 * cin, :], w1_ref[3 * cin:4 * cin, :],
         w1_ref[5 * cin:6 * cin, :]], axis=-1)
    s_taps = jnp.sum(w1af, axis=0, keepdims=True)

    gw_col = glow_ref[:, 0:1]
    gb_col = glow_ref[:, 1:2]
    tcol = lax.broadcasted_iota(jnp.int32, (t_len, 1), 0)
    first_t = tcol == 0
    last_t = tcol == (t_len - 1)
    cm = jnp.where(first_t, 0.0, pltpu.roll(gw_col, 1, axis=0))
    cp = jnp.where(last_t, 0.0, pltpu.roll(gw_col, t_len - 1, axis=0))
    dm = jnp.where(first_t, 0.0, pltpu.roll(gb_col, 1, axis=0))
    dp = jnp.where(last_t, 0.0, pltpu.roll(gb_col, t_len - 1, axis=0))
    g_const = (dm * s_taps[:, 0:hid] + gb_col * s_taps[:, hid:2 * hid]
               + dp * s_taps[:, 2 * hid:] + b12_ref[0:1, :])
    g16 = g_const.astype(jnp.bfloat16)
    cm16 = cm.astype(jnp.bfloat16)
    c016 = gw_col.astype(jnp.bfloat16)
    cp16 = cp.astype(jnp.bfloat16)

    a_b = jnp.dot(af_ref[...].astype(jnp.bfloat16), wm_ref[...].astype(jnp.bfloat16),
                  preferred_element_type=jnp.float32) + bm_ref[...]
    q = jnp.dot(a_b.astype(jnp.bfloat16), w1af.astype(jnp.bfloat16),
                preferred_element_type=jnp.float32)
    q16 = q.astype(jnp.bfloat16)

    for c in range(n_chunks):
        rows = pl.ds(c * rc, rc)

        qc = q16[c * nbc:(c + 1) * nbc]
        f_cond = (qc[:, None, 0:hid] * cm16[None]
                  + qc[:, None, hid:2 * hid] * c016[None]
                  + qc[:, None, 2 * hid:] * cp16[None]
                  + g16[None]).reshape(rc, hid)

        z = z1_ref[rows, :].astype(jnp.bfloat16)
        z_prev = jnp.where(is_first, 0, pltpu.roll(z, 1, axis=0))
        z_next = jnp.where(is_last, 0, pltpu.roll(z, rc - 1, axis=0))
        zs = jnp.concatenate([z_prev, z, z_next], axis=-1)
        y = jnp.dot(zs, w1z, preferred_element_type=jnp.float32)
        y = jnp.maximum(y.astype(jnp.bfloat16) + f_cond, 0)

        y = jnp.dot(y, w2, preferred_element_type=jnp.float32)
        y = jnp.maximum(y.astype(jnp.bfloat16) + b12[1:2, :], 0)

        p = jnp.dot(y, w3, preferred_element_type=jnp.float32)
        p_prev = pltpu.roll(p[:, :cout], 1, axis=0)
        p_next = pltpu.roll(p[:, 2 * cout:], rc - 1, axis=0)
        out = (p[:, cout:2 * cout]
               + jnp.where(is_first, 0.0, p_prev)
               + jnp.where(is_last, 0.0, p_next)
               + b3_ref[...])
        o_ref[rows, :] = out


@jax.jit
def kernel(z1, audio_features, w_mlp_t, b_mlp, glow, w1s, w2m, b12, w3s, b3):
    N, T, Cin = z1.shape
    Dc = audio_features.shape[1]
    Cout = b3.shape[1]

    max_rows = 8192
    nb = N
    if N * T > max_rows:
        for cand in range(min(N, max(1, max_rows // T)), 0, -1):
            if N % cand == 0 and cand % 8 == 0:
                nb = cand
                break
    grid = (N // nb,)
    R = nb * T
    n_chunks = max(1, R // 512)
    while nb % n_chunks != 0:
        n_chunks //= 2

    z1_flat = z1.reshape(N * T, Cin)

    plist = [w_mlp_t, b_mlp, glow, w1s, w2m, b12, w3s, b3]

    in_specs = [
        pl.BlockSpec((nb, Dc), lambda g: (g, 0)),
        pl.BlockSpec((R, Cin), lambda g: (g, 0)),
    ] + [pl.BlockSpec(p.shape, lambda g: (0, 0)) for p in plist]

    out = pl.pallas_call(
        functools.partial(_fused_kernel, t_len=T, n_chunks=n_chunks),
        out_shape=jax.ShapeDtypeStruct((N * T, Cout), jnp.float32),
        grid=grid,
        in_specs=in_specs,
        out_specs=pl.BlockSpec((R, Cout), lambda g: (g, 0)),
        compiler_params=pltpu.CompilerParams(
            dimension_semantics=("parallel",),
            vmem_limit_bytes=64 * 1024 * 1024),
    )(audio_features, z1_flat, *plist)
    return out.reshape(N, T, Cout)


# speedup vs baseline: 1.2908x; 1.2908x over previous
import functools

import jax
import jax.numpy as jnp
from jax import lax
from jax.experimental import pallas as pl
from jax.experimental.pallas import tpu as pltpu


def _fused_kernel(
    af_ref,
    z1_ref,
    wm_ref,
    bm_ref,
    glow_ref,
    w1_ref,
    w2_ref,
    b12_ref,
    w3_ref,
    b3_ref,
    o_ref,
    *,
    t_len,
    n_chunks,
):
    R = z1_ref.shape[0]
    cin = z1_ref.shape[1]
    hid = w2_ref.shape[0]
    cout = o_ref.shape[1]
    rc = R // n_chunks
    nbc = rc // t_len

    t_idx = lax.broadcasted_iota(jnp.int32, (rc, 1), 0) % t_len
    is_first = t_idx == 0
    is_last = t_idx == (t_len - 1)

    w1 = w1_ref[...].astype(jnp.bfloat16)
    w2 = w2_ref[...].astype(jnp.bfloat16)
    w3 = jnp.concatenate(
        [w3_ref[0:hid, :], w3_ref[hid:2 * hid, :], w3_ref[2 * hid:, :]],
        axis=-1).astype(jnp.bfloat16)
    b12 = b12_ref[...].astype(jnp.bfloat16)

    tcol = lax.broadcasted_iota(jnp.int32, (t_len, 1), 0)
    gw_col = glow_ref[:, 0:1]
    gb_col = glow_ref[:, 1:2]
    zcol = jnp.zeros_like(gw_col)
    cm = jnp.where(tcol == 0, zcol, pltpu.roll(gw_col, 1, axis=0))
    dm = jnp.where(tcol == 0, zcol, pltpu.roll(gb_col, 1, axis=0))
    cp = jnp.where(tcol == t_len - 1, zcol, pltpu.roll(gw_col, t_len - 1, axis=0))
    dp = jnp.where(tcol == t_len - 1, zcol, pltpu.roll(gb_col, t_len - 1, axis=0))
    cm16, dm16 = cm.astype(jnp.bfloat16), dm.astype(jnp.bfloat16)
    c016, d016 = gw_col.astype(jnp.bfloat16), gb_col.astype(jnp.bfloat16)
    cp16, dp16 = cp.astype(jnp.bfloat16), dp.astype(jnp.bfloat16)

    a_b = jnp.dot(af_ref[...].astype(jnp.bfloat16), wm_ref[...].astype(jnp.bfloat16),
                  preferred_element_type=jnp.float32) + bm_ref[...]
    a_b16 = a_b.astype(jnp.bfloat16)

    for c in range(n_chunks):
        rows = pl.ds(c * rc, rc)

        ab3 = jnp.broadcast_to(a_b16[c * nbc:(c + 1) * nbc][:, None, :],
                               (nbc, t_len, cin))
        afg_m = (ab3 * cm16[None] + dm16[None]).reshape(rc, cin)
        afg_0 = (ab3 * c016[None] + d016[None]).reshape(rc, cin)
        afg_p = (ab3 * cp16[None] + dp16[None]).reshape(rc, cin)

        z1f = z1_ref[rows, :]
        z1c = z1f.astype(jnp.bfloat16)
        if c > 0:
            z1m_src = z1_ref[pl.ds(c * rc - 1, rc), :]
        else:
            z1m_src = pltpu.roll(z1f, 1, axis=0)
        if c < n_chunks - 1:
            z1p_src = z1_ref[pl.ds(c * rc + 1, rc), :]
        else:
            z1p_src = pltpu.roll(z1f, rc - 1, axis=0)
        z1m = jnp.where(is_first, 0, z1m_src.astype(jnp.bfloat16))
        z1p = jnp.where(is_last, 0, z1p_src.astype(jnp.bfloat16))

        zs = jnp.concatenate([z1m, afg_m, z1c, afg_0, z1p, afg_p],
                             axis=-1)
        y = jnp.dot(zs, w1, preferred_element_type=jnp.float32)
        y = jnp.maximum(y.astype(jnp.bfloat16) + b12[0:1, :], 0)

        y = jnp.dot(y, w2, preferred_element_type=jnp.float32)
        y = jnp.maximum(y.astype(jnp.bfloat16) + b12[1:2, :], 0)

        p = jnp.dot(y, w3, preferred_element_type=jnp.float32)
        p_prev = pltpu.roll(p[:, :cout], 1, axis=0)
        p_next = pltpu.roll(p[:, 2 * cout:], rc - 1, axis=0)
        out = (p[:, cout:2 * cout]
               + jnp.where(is_first, 0.0, p_prev)
               + jnp.where(is_last, 0.0, p_next)
               + b3_ref[...])
        o_ref[rows, :] = out


@jax.jit
def kernel(z1, audio_features, w_mlp_t, b_mlp, glow, w1s, w2m, b12, w3s, b3):
    N, T, Cin = z1.shape
    Dc = audio_features.shape[1]
    Cout = b3.shape[1]

    max_rows = 8192
    nb = N
    if N * T > max_rows:
        for cand in range(min(N, max(1, max_rows // T)), 0, -1):
            if N % cand == 0 and cand % 8 == 0:
                nb = cand
                break
    grid = (N // nb,)
    R = nb * T
    n_chunks = max(1, R // 512)
    while nb % n_chunks != 0:
        n_chunks //= 2

    z1_flat = z1.reshape(N * T, Cin)

    plist = [w_mlp_t, b_mlp, glow, w1s, w2m, b12, w3s, b3]

    in_specs = [
        pl.BlockSpec((nb, Dc), lambda g: (g, 0)),
        pl.BlockSpec((R, Cin), lambda g: (g, 0)),
    ] + [pl.BlockSpec(p.shape, lambda g: (0, 0)) for p in plist]

    out = pl.pallas_call(
        functools.partial(_fused_kernel, t_len=T, n_chunks=n_chunks),
        out_shape=jax.ShapeDtypeStruct((N * T, Cout), jnp.float32),
        grid=grid,
        in_specs=in_specs,
        out_specs=pl.BlockSpec((R, Cout), lambda g: (g, 0)),
        compiler_params=pltpu.CompilerParams(
            dimension_semantics=("parallel",),
            vmem_limit_bytes=64 * 1024 * 1024),
    )(audio_features, z1_flat, *plist)
    return out.reshape(N, T, Cout)
